# Initial kernel scaffold; baseline (speedup 1.0000x reference)
import jax
import jax.numpy as jnp
from jax.experimental import pallas as pl


def kernel(x, facs, noise):
    raise NotImplementedError("write your pallas kernel here")


# speedup vs baseline: 11.9115x; 11.9115x over previous
import functools

import jax
import jax.numpy as jnp
from jax.experimental import pallas as pl
from jax.experimental.pallas import tpu as pltpu


def _pool_body(x_ref, pooled_ref, *, w):
    v = x_ref[...].astype(jnp.float32)
    top = v[:, 0:w]
    bot = v[:, w:2 * w]
    rs = top + bot
    rm = jnp.maximum(top, bot)
    i = jax.lax.broadcasted_iota(jnp.int32, (w, w // 2), 0)
    j = jax.lax.broadcasted_iota(jnp.int32, (w, w // 2), 1)
    e0 = (i == 2 * j).astype(jnp.float32)
    e1 = (i == 2 * j + 1).astype(jnp.float32)

    def dot(a, b):
        return jax.lax.dot_general(
            a, b, (((1,), (0,)), ((), ())),
            precision=jax.lax.Precision.HIGHEST,
            preferred_element_type=jnp.float32)

    cs = dot(rs, e0 + e1)
    cm = jnp.maximum(dot(rm, e0), dot(rm, e1))
    pooled_ref[...] = (cs * 0.25 + cm) * 0.5


def _noise_body(facs_ref, pooled_ref, noise_ref, o_ref, *, block):
    i = pl.program_id(0)
    pooled = pooled_ref[...]
    for b in range(block):
        fac = facs_ref[i * block + b]
        o_ref[b] = (pooled + fac * noise_ref[b].astype(jnp.float32)).astype(
            o_ref.dtype)


def kernel(x, facs, noise):
    N, C, H, W = x.shape
    cutn, _, cs, _ = noise.shape
    rows = C * cs
    x2 = x[0].reshape(rows, 2 * W)
    pooled = pl.pallas_call(
        functools.partial(_pool_body, w=W),
        out_shape=jax.ShapeDtypeStruct((rows, cs), jnp.float32),
        grid=(2,),
        in_specs=[pl.BlockSpec((rows // 2, 2 * W), lambda r: (r, 0))],
        out_specs=pl.BlockSpec((rows // 2, cs), lambda r: (r, 0)),
        compiler_params=pltpu.CompilerParams(
            dimension_semantics=("parallel",)),
    )(x2).reshape(C, cs, cs)

    B = 4
    out = pl.pallas_call(
        functools.partial(_noise_body, block=B),
        out_shape=jax.ShapeDtypeStruct((cutn, C, cs, cs), x.dtype),
        grid=(cutn // B,),
        in_specs=[
            pl.BlockSpec(memory_space=pltpu.MemorySpace.SMEM),
            pl.BlockSpec((C, cs, cs), lambda i: (0, 0, 0)),
            pl.BlockSpec((B, C, cs, cs), lambda i: (i, 0, 0, 0)),
        ],
        out_specs=pl.BlockSpec((B, C, cs, cs), lambda i: (i, 0, 0, 0)),
        compiler_params=pltpu.CompilerParams(
            dimension_semantics=("parallel",),
            vmem_limit_bytes=32 * 1024 * 1024,
        ),
    )(facs, pooled, noise)

    return out
